# baseline (speedup 1.0000x reference)
import functools
import jax
import jax.numpy as jnp
from jax import lax
from jax.experimental import pallas as pl
from jax.experimental.pallas import tpu as pltpu
from jax.experimental.pallas import tpu_sc as plsc

NC = 2
NS = 16
LANES = 16

N = 50000
E = 800000
H = 64

HALF = N // NC
ROWS_PER_TILE = 1568
HALF_PAD = ROWS_PER_TILE * NS

BLK = 128
NBLKS = E // BLK
BH, BHR = NBLKS // (NC * NS), NBLKS % (NC * NS)
BA, BAR = NBLKS // NS, NBLKS % NS

DEG_PER_TILE = 3136
NP = DEG_PER_TILE * NS
ZROW = N


def _zero_1d(ref, nwords):
    z = jnp.zeros((LANES,), ref.dtype)

    def body(i, _):
        ref[pl.ds(i * LANES, LANES)] = z
        return 0

    lax.fori_loop(0, nwords // LANES, body, 0)


def _zero_2d(ref, nrows, ncols):
    z = jnp.zeros((LANES,), ref.dtype)

    def body(i, _):
        r = i // (ncols // LANES)
        l = i % (ncols // LANES)
        ref[r, pl.ds(l * LANES, LANES)] = z
        return 0

    lax.fori_loop(0, nrows * (ncols // LANES), body, 0)


@functools.lru_cache(maxsize=1)
def _sc_kernels():
    mesh = plsc.VectorSubcoreMesh(core_axis_name="c", subcore_axis_name="s",
                                  num_cores=NC, num_subcores=NS)

    def _ds8(off, size):
        return pl.ds(pl.multiple_of(off, 8), size)

    @functools.partial(
        pl.kernel,
        out_type=jax.ShapeDtypeStruct((NC * NP,), jnp.float32),
        mesh=mesh,
        scratch_types=[
            pltpu.VMEM((BLK,), jnp.int32),
            pltpu.VMEM((BLK,), jnp.float32),
            pltpu.VMEM((DEG_PER_TILE,), jnp.float32),
            pltpu.VMEM_SHARED((NP,), jnp.float32),
        ],
    )
    def sc_degree(edge_hbm, out_hbm, colb, ones, zbuf, hist_sh):
        c = lax.axis_index("c")
        s = lax.axis_index("s")
        one = jnp.ones((LANES,), jnp.float32)
        for i in range(BLK // LANES):
            ones[pl.ds(i * LANES, LANES)] = one
        _zero_1d(zbuf, DEG_PER_TILE)
        pltpu.sync_copy(zbuf, hist_sh.at[_ds8(s * DEG_PER_TILE, DEG_PER_TILE)])
        plsc.subcore_barrier()

        w = c * NS + s
        st = w * BH + jnp.minimum(w, BHR)
        nb = BH + jnp.where(w < BHR, 1, 0)

        def blk(j, _):
            off = (st + j) * BLK
            pltpu.sync_copy(edge_hbm.at[_ds8(E + off, BLK)], colb)
            pltpu.sync_copy(ones, hist_sh.at[colb], add=True)
            return 0

        lax.fori_loop(0, nb, blk, 0)
        plsc.subcore_barrier()
        pltpu.sync_copy(hist_sh.at[_ds8(s * DEG_PER_TILE, DEG_PER_TILE)], zbuf)
        pltpu.sync_copy(zbuf, out_hbm.at[_ds8(c * NP + s * DEG_PER_TILE,
                                              DEG_PER_TILE)])

    @functools.partial(
        pl.kernel,
        out_type=jax.ShapeDtypeStruct((NC * NP,), jnp.float32),
        mesh=mesh,
        scratch_types=[
            pltpu.VMEM((BLK,), jnp.int32),
            pltpu.VMEM((BLK,), jnp.int32),
            pltpu.VMEM((BLK,), jnp.float32),
            pltpu.VMEM((DEG_PER_TILE,), jnp.float32),
            pltpu.VMEM_SHARED((NP,), jnp.float32),
            pltpu.SemaphoreType.DMA,
        ],
    )
    def sc_sscatter(edge_hbm, dis_hbm, out_hbm, rowb, colb, disb, zbuf,
                    hist_sh, sem):
        c = lax.axis_index("c")
        s = lax.axis_index("s")
        _zero_1d(zbuf, DEG_PER_TILE)
        pltpu.sync_copy(zbuf, hist_sh.at[_ds8(s * DEG_PER_TILE, DEG_PER_TILE)])
        plsc.subcore_barrier()

        w = c * NS + s
        st = w * BH + jnp.minimum(w, BHR)
        nb = BH + jnp.where(w < BHR, 1, 0)

        def blk(j, _):
            off = (st + j) * BLK
            pltpu.sync_copy(edge_hbm.at[_ds8(off, BLK)], rowb)
            pltpu.sync_copy(edge_hbm.at[_ds8(E + off, BLK)], colb)
            pltpu.async_copy(dis_hbm.at[rowb], disb, sem).wait()
            pltpu.sync_copy(disb, hist_sh.at[colb], add=True)
            return 0

        lax.fori_loop(0, nb, blk, 0)
        plsc.subcore_barrier()
        pltpu.sync_copy(hist_sh.at[_ds8(s * DEG_PER_TILE, DEG_PER_TILE)], zbuf)
        pltpu.sync_copy(zbuf, out_hbm.at[_ds8(c * NP + s * DEG_PER_TILE,
                                              DEG_PER_TILE)])

    @functools.partial(
        pl.kernel,
        out_type=jax.ShapeDtypeStruct((NC, HALF_PAD, H), jnp.float32),
        mesh=mesh,
        scratch_types=[
            pltpu.VMEM((BLK,), jnp.int32),
            pltpu.VMEM((BLK,), jnp.int32),
            pltpu.VMEM((BLK,), jnp.int32),
            pltpu.VMEM((BLK,), jnp.int32),
            pltpu.VMEM((BLK, H), jnp.float32),
            pltpu.VMEM_SHARED((HALF_PAD, H), jnp.float32),
            pltpu.SemaphoreType.DMA,
        ],
        compiler_params=pltpu.CompilerParams(use_tc_tiling_on_sc=False),
    )
    def sc_edge_sum(edge_hbm, z_hbm, out_hbm, colb, rowb, ccol, crow,
                    rowsbuf, acc_sh, sem):
        c = lax.axis_index("c")
        s = lax.axis_index("s")
        base = c * HALF

        _zero_2d(rowsbuf, BLK, H)
        row0 = s * ROWS_PER_TILE
        for k in range(ROWS_PER_TILE // BLK):
            pltpu.sync_copy(rowsbuf, acc_sh.at[_ds8(row0 + k * BLK, BLK), :])
        rem = ROWS_PER_TILE % BLK
        if rem:
            pltpu.sync_copy(
                rowsbuf.at[pl.ds(0, rem), :],
                acc_sh.at[_ds8(row0 + (ROWS_PER_TILE // BLK) * BLK, rem), :])
        plsc.subcore_barrier()

        st = s * BA + jnp.minimum(s, BAR)
        nb = BA + jnp.where(s < BAR, 1, 0)
        zero16 = jnp.zeros((LANES,), jnp.int32)

        def blk(j, _):
            off = (st + j) * BLK
            pltpu.sync_copy(edge_hbm.at[_ds8(E + off, BLK)], colb)
            pltpu.sync_copy(edge_hbm.at[_ds8(off, BLK)], rowb)
            for g in range(BLK // LANES):
                cv = colb[pl.ds(g * LANES, LANES)]
                rv = rowb[pl.ds(g * LANES, LANES)]
                inb = (cv >= base) & (cv < base + HALF)
                crow[pl.ds(g * LANES, LANES)] = jnp.where(inb, rv,
                                                          zero16 + ZROW)
                ccol[pl.ds(g * LANES, LANES)] = jnp.where(inb, cv - base,
                                                          zero16)
            pltpu.async_copy(z_hbm.at[crow], rowsbuf, sem).wait()
            pltpu.sync_copy(rowsbuf, acc_sh.at[ccol], add=True)
            return 0

        lax.fori_loop(0, nb, blk, 0)
        plsc.subcore_barrier()
        for k in range(ROWS_PER_TILE // BLK):
            pltpu.sync_copy(acc_sh.at[_ds8(row0 + k * BLK, BLK), :], rowsbuf)
            pltpu.sync_copy(rowsbuf,
                            out_hbm.at[c, _ds8(row0 + k * BLK, BLK), :])
        if rem:
            r0 = row0 + (ROWS_PER_TILE // BLK) * BLK
            pltpu.sync_copy(acc_sh.at[_ds8(r0, rem), :],
                            rowsbuf.at[pl.ds(0, rem), :])
            pltpu.sync_copy(rowsbuf.at[pl.ds(0, rem), :],
                            out_hbm.at[c, _ds8(r0, rem), :])

    return sc_degree, sc_sscatter, sc_edge_sum


NG = 25
BRS = N // NG

def _row_spec(w):
    return pl.BlockSpec((BRS, w), lambda i: (i, 0))


def _full_spec(shape):
    return pl.BlockSpec(shape, lambda i: (0, 0))


def _tc_pool_body(x_ref, d_ref, win_ref, bin_ref, h_ref, acc):
    i = pl.program_id(0)

    @pl.when(i == 0)
    def _():
        acc[...] = jnp.zeros_like(acc)
        acc[1:2, :] = jnp.full((1, H), -jnp.inf, jnp.float32)

    x = x_ref[...]
    rn = jnp.sqrt(jnp.sum(x * x, axis=1, keepdims=True))
    x = x / jnp.maximum(rn, 1e-12) + d_ref[0, 0]
    acc[0:1, :] = acc[0:1, :] + jnp.sum(x, axis=0, keepdims=True)
    acc[1:2, :] = jnp.maximum(acc[1:2, :], jnp.max(x, axis=0, keepdims=True))
    acc[2:3, :] = acc[2:3, :] + jnp.sum(x * x, axis=0, keepdims=True)

    @pl.when(i == NG - 1)
    def _():
        mean = acc[0:1, :] / N
        mx = acc[1:2, :]
        var1 = (acc[2:3, :] - N * mean * mean) / (N - 1)
        pooled = jnp.concatenate([mean, mx, jnp.sqrt(var1)], axis=1)
        h_ref[...] = jax.nn.relu(
            jnp.dot(pooled, win_ref[...], preferred_element_type=jnp.float32)
            + bin_ref[...])


def _call_pool(x, delta, W_in, b_in, interpret=False):
    return pl.pallas_call(
        _tc_pool_body,
        grid=(NG,),
        in_specs=[_row_spec(H), _full_spec((1, 1)), _full_spec((3 * H, H)),
                  _full_spec((1, H))],
        out_specs=_full_spec((1, H)),
        out_shape=jax.ShapeDtypeStruct((1, H), jnp.float32),
        scratch_shapes=[pltpu.VMEM((8, H), jnp.float32)],
        interpret=interpret,
    )(x, delta, W_in, b_in)


def _tc_dis_body(degp_ref, dis_ref):
    d = degp_ref[...]
    dis_ref[...] = lax.rsqrt(1.0 + d[0:1, :] + d[1:2, :])


def _call_dis(deg_part, interpret=False):
    return pl.pallas_call(
        _tc_dis_body,
        out_shape=jax.ShapeDtypeStruct((1, NP), jnp.float32),
        interpret=interpret,
    )(deg_part)


def _tc_l1stats_body(dis_ref, s_ref, h_ref, wm0_ref, g0_ref, b0_ref,
                     ss_ref, acc):
    i = pl.program_id(0)

    @pl.when(i == 0)
    def _():
        acc[...] = jnp.zeros_like(acc)

    dis = dis_ref[...]
    cvec = dis * (s_ref[...] + dis)
    acc[0:1, 0:1] = acc[0:1, 0:1] + jnp.sum(cvec).reshape(1, 1)
    acc[1:2, 0:1] = acc[1:2, 0:1] + jnp.sum(cvec * cvec).reshape(1, 1)

    @pl.when(i == NG - 1)
    def _():
        mc = acc[0, 0] / N
        vc = acc[1, 0] / N - mc * mc
        m = jnp.dot(h_ref[...], wm0_ref[...],
                    preferred_element_type=jnp.float32)
        scale = m / jnp.sqrt(vc * m * m + 1e-5) * g0_ref[...]
        shift = b0_ref[...] - mc * scale
        ss_ref[0:1, :] = scale
        ss_ref[1:2, :] = shift


def _call_l1stats(dis2d, svec, h, Wm0, g0, b0, interpret=False):
    return pl.pallas_call(
        _tc_l1stats_body,
        grid=(NG,),
        in_specs=[_row_spec(1), _row_spec(1), _full_spec((1, H)),
                  _full_spec((H, H)), _full_spec((1, H)), _full_spec((1, H))],
        out_specs=_full_spec((8, H)),
        out_shape=jax.ShapeDtypeStruct((8, H), jnp.float32),
        scratch_shapes=[pltpu.VMEM((8, H), jnp.float32)],
        interpret=interpret,
    )(dis2d, svec, h, Wm0, g0, b0)


def _tc_l1apply_body(dis_ref, s_ref, ss_ref, wm1_ref, x1_ref, z2_ref):
    dis = dis_ref[...]
    cvec = dis * (s_ref[...] + dis)
    x1 = jax.nn.relu(cvec * ss_ref[0:1, :] + ss_ref[1:2, :])
    x1_ref[...] = x1
    z2_ref[...] = jnp.dot(x1, wm1_ref[...],
                          preferred_element_type=jnp.float32) * dis


def _call_l1apply(dis2d, svec, ss, Wm1, interpret=False):
    return pl.pallas_call(
        _tc_l1apply_body,
        grid=(NG,),
        in_specs=[_row_spec(1), _row_spec(1), _full_spec((8, H)),
                  _full_spec((H, H))],
        out_specs=[_row_spec(H), _row_spec(H)],
        out_shape=[jax.ShapeDtypeStruct((N, H), jnp.float32),
                   jax.ShapeDtypeStruct((N, H), jnp.float32)],
        interpret=interpret,
    )(dis2d, svec, ss, Wm1)


def _tc_lstats_body(x_ref, es_ref, z_ref, dis_ref, ws_ref, cb_ref,
                    out_ref, st_ref, acc):
    i = pl.program_id(0)

    @pl.when(i == 0)
    def _():
        acc[...] = jnp.zeros_like(acc)

    dis = dis_ref[...]
    agg = dis * (es_ref[...] + z_ref[...])
    out = agg + jnp.dot(x_ref[...], ws_ref[...],
                        preferred_element_type=jnp.float32) + cb_ref[...]
    out_ref[...] = out
    acc[0:1, :] = acc[0:1, :] + jnp.sum(out, axis=0, keepdims=True)
    acc[1:2, :] = acc[1:2, :] + jnp.sum(out * out, axis=0, keepdims=True)

    @pl.when(i == NG - 1)
    def _():
        st_ref[...] = acc[...]


def _call_lstats(x, esum, z, dis2d, Ws, cb, interpret=False):
    return pl.pallas_call(
        _tc_lstats_body,
        grid=(NG,),
        in_specs=[_row_spec(H), _row_spec(H), _row_spec(H), _row_spec(1),
                  _full_spec((H, H)), _full_spec((1, H))],
        out_specs=[_row_spec(H), _full_spec((8, H))],
        out_shape=[jax.ShapeDtypeStruct((N, H), jnp.float32),
                   jax.ShapeDtypeStruct((8, H), jnp.float32)],
        scratch_shapes=[pltpu.VMEM((8, H), jnp.float32)],
        interpret=interpret,
    )(x, esum, z, dis2d, Ws, cb)


def _tc_lapply_body(x_ref, out_ref, st_ref, g_ref, b_ref, dis_ref, wmn_ref,
                    xo_ref, zo_ref):
    mean = st_ref[0:1, :] / N
    var = st_ref[1:2, :] / N - mean * mean
    bn = (out_ref[...] - mean) / jnp.sqrt(var + 1e-5) * g_ref[...] + b_ref[...]
    xn = jax.nn.relu(bn) + x_ref[...]
    xo_ref[...] = xn
    zo_ref[...] = jnp.dot(xn, wmn_ref[...],
                          preferred_element_type=jnp.float32) * dis_ref[...]


def _call_lapply(x, out, st, g, b, dis2d, Wmn, interpret=False):
    return pl.pallas_call(
        _tc_lapply_body,
        grid=(NG,),
        in_specs=[_row_spec(H), _row_spec(H), _full_spec((8, H)),
                  _full_spec((1, H)), _full_spec((1, H)), _row_spec(1),
                  _full_spec((H, H))],
        out_specs=[_row_spec(H), _row_spec(H)],
        out_shape=[jax.ShapeDtypeStruct((N, H), jnp.float32),
                   jax.ShapeDtypeStruct((N, H), jnp.float32)],
        interpret=interpret,
    )(x, out, st, g, b, dis2d, Wmn)


def _tc_head_body(x_ref, out_ref, st_ref, g_ref, b_ref, wo_ref, bo_ref,
                  pe_ref):
    mean = st_ref[0:1, :] / N
    var = st_ref[1:2, :] / N - mean * mean
    bn = (out_ref[...] - mean) / jnp.sqrt(var + 1e-5) * g_ref[...] + b_ref[...]
    xn = jax.nn.relu(bn) + x_ref[...]
    pe = jnp.dot(xn, wo_ref[...],
                 preferred_element_type=jnp.float32) + bo_ref[...]
    rn = jnp.sqrt(jnp.sum(pe * pe, axis=1, keepdims=True))
    pe_ref[...] = pe / jnp.maximum(rn, 1e-12)


def _call_head(x, out, st, g, b, W_out, b_out2d, interpret=False):
    P = W_out.shape[1]
    return pl.pallas_call(
        _tc_head_body,
        grid=(NG,),
        in_specs=[_row_spec(H), _row_spec(H), _full_spec((8, H)),
                  _full_spec((1, H)), _full_spec((1, H)),
                  _full_spec((H, P)), _full_spec((1, P))],
        out_specs=_row_spec(P),
        out_shape=jax.ShapeDtypeStruct((N, P), jnp.float32),
        interpret=interpret,
    )(x, out, st, g, b, W_out, b_out2d)


_ZPAD = None


def _pad_z(z):
    return jnp.concatenate(
        [z, jnp.zeros((NP - N, H), jnp.float32)], axis=0)


def kernel(edge_index, num_nodes, x_init, W_in, b_in, W_msg, W_self, conv_b,
           bn_g, bn_b, W_out, b_out):
    f32 = jnp.float32
    sc_degree, sc_sscatter, sc_edge_sum = _sc_kernels()
    delta = (jnp.asarray(num_nodes) - N).astype(f32).reshape(1, 1)

    h = _call_pool(x_init, delta, W_in, b_in.reshape(1, H))

    eflat = edge_index.reshape(2 * E)
    deg_part = sc_degree(eflat).reshape(NC, NP)
    dis_flat = _call_dis(deg_part).reshape(NP)
    s_part = sc_sscatter(eflat, dis_flat).reshape(NC, NP)

    dis2d = dis_flat[:N, None]
    svec = (s_part[0, :N] + s_part[1, :N])[:, None]

    ss = _call_l1stats(dis2d, svec, h, W_msg[0], bn_g[0].reshape(1, H),
                       bn_b[0].reshape(1, H))
    x1, z2 = _call_l1apply(dis2d, svec, ss, W_msg[1])

    es2 = sc_edge_sum(eflat, _pad_z(z2))
    esum2 = jnp.concatenate([es2[0, :HALF], es2[1, :HALF]], axis=0)

    out2, st2 = _call_lstats(x1, esum2, z2, dis2d, W_self[1],
                             conv_b[1].reshape(1, H))
    x2, z3 = _call_lapply(x1, out2, st2, bn_g[1].reshape(1, H),
                          bn_b[1].reshape(1, H), dis2d, W_msg[2])

    es3 = sc_edge_sum(eflat, _pad_z(z3))
    esum3 = jnp.concatenate([es3[0, :HALF], es3[1, :HALF]], axis=0)

    out3, st3 = _call_lstats(x2, esum3, z3, dis2d, W_self[2],
                             conv_b[2].reshape(1, H))
    pe = _call_head(x2, out3, st3, bn_g[2].reshape(1, H),
                    bn_b[2].reshape(1, H), W_out,
                    b_out.reshape(1, W_out.shape[1]))
    return pe


# speedup vs baseline: 12.9196x; 12.9196x over previous
import functools
import jax
import jax.numpy as jnp
from jax import lax
from jax.experimental import pallas as pl
from jax.experimental.pallas import tpu as pltpu
from jax.experimental.pallas import tpu_sc as plsc

NC = 2
NS = 16
LANES = 16

N = 50000
E = 800000
H = 64

HALF = N // NC
ROWS_PER_TILE = 1568
HALF_PAD = ROWS_PER_TILE * NS

BLK = 128
SUP = 8
NBLKS = 6272
E_PAD = NBLKS * BLK
BH = NBLKS // (NC * NS)
BA = NBLKS // NS
NSUP = BA // SUP

DEG_PER_TILE = 3136
NP = DEG_PER_TILE * NS
ZROW = N


def _zero_1d(ref, nwords):
    z = jnp.zeros((LANES,), ref.dtype)

    def body(i, _):
        ref[pl.ds(i * LANES, LANES)] = z
        return 0

    lax.fori_loop(0, nwords // LANES, body, 0)


def _zero_2d(ref, nrows, ncols):
    z = jnp.zeros((LANES,), ref.dtype)

    def body(i, _):
        r = i // (ncols // LANES)
        l = i % (ncols // LANES)
        ref[r, pl.ds(l * LANES, LANES)] = z
        return 0

    lax.fori_loop(0, nrows * (ncols // LANES), body, 0)


@functools.lru_cache(maxsize=1)
def _sc_kernels():
    mesh = plsc.VectorSubcoreMesh(core_axis_name="c", subcore_axis_name="s",
                                  num_cores=NC, num_subcores=NS)

    def _ds8(off, size):
        return pl.ds(pl.multiple_of(off, 8), size)

    @functools.partial(
        pl.kernel,
        out_type=jax.ShapeDtypeStruct((NC * NP,), jnp.float32),
        mesh=mesh,
        scratch_types=[
            pltpu.VMEM((BLK,), jnp.int32),
            pltpu.VMEM((BLK,), jnp.float32),
            pltpu.VMEM((DEG_PER_TILE,), jnp.float32),
            pltpu.VMEM_SHARED((NP,), jnp.float32),
        ],
    )
    def sc_degree(edge_hbm, out_hbm, colb, ones, zbuf, hist_sh):
        c = lax.axis_index("c")
        s = lax.axis_index("s")
        one = jnp.ones((LANES,), jnp.float32)
        for i in range(BLK // LANES):
            ones[pl.ds(i * LANES, LANES)] = one
        _zero_1d(zbuf, DEG_PER_TILE)
        pltpu.sync_copy(zbuf, hist_sh.at[_ds8(s * DEG_PER_TILE, DEG_PER_TILE)])
        plsc.subcore_barrier()

        w = c * NS + s
        st = w * BH

        def blk(j, _):
            off = (st + j) * BLK
            pltpu.sync_copy(edge_hbm.at[_ds8(E_PAD + off, BLK)], colb)
            pltpu.sync_copy(ones, hist_sh.at[colb], add=True)
            return 0

        lax.fori_loop(0, BH, blk, 0)
        plsc.subcore_barrier()
        pltpu.sync_copy(hist_sh.at[_ds8(s * DEG_PER_TILE, DEG_PER_TILE)], zbuf)
        pltpu.sync_copy(zbuf, out_hbm.at[_ds8(c * NP + s * DEG_PER_TILE,
                                              DEG_PER_TILE)])

    @functools.partial(
        pl.kernel,
        out_type=jax.ShapeDtypeStruct((NC * NP,), jnp.float32),
        mesh=mesh,
        scratch_types=[
            pltpu.VMEM((BLK,), jnp.int32),
            pltpu.VMEM((BLK,), jnp.int32),
            pltpu.VMEM((BLK,), jnp.float32),
            pltpu.VMEM((DEG_PER_TILE,), jnp.float32),
            pltpu.VMEM_SHARED((NP,), jnp.float32),
            pltpu.SemaphoreType.DMA,
        ],
    )
    def sc_sscatter(edge_hbm, dis_hbm, out_hbm, rowb, colb, disb, zbuf,
                    hist_sh, sem):
        c = lax.axis_index("c")
        s = lax.axis_index("s")
        _zero_1d(zbuf, DEG_PER_TILE)
        pltpu.sync_copy(zbuf, hist_sh.at[_ds8(s * DEG_PER_TILE, DEG_PER_TILE)])
        plsc.subcore_barrier()

        w = c * NS + s
        st = w * BH

        def blk(j, _):
            off = (st + j) * BLK
            pltpu.sync_copy(edge_hbm.at[_ds8(off, BLK)], rowb)
            pltpu.sync_copy(edge_hbm.at[_ds8(E_PAD + off, BLK)], colb)
            pltpu.async_copy(dis_hbm.at[rowb], disb, sem).wait()
            pltpu.sync_copy(disb, hist_sh.at[colb], add=True)
            return 0

        lax.fori_loop(0, BH, blk, 0)
        plsc.subcore_barrier()
        pltpu.sync_copy(hist_sh.at[_ds8(s * DEG_PER_TILE, DEG_PER_TILE)], zbuf)
        pltpu.sync_copy(zbuf, out_hbm.at[_ds8(c * NP + s * DEG_PER_TILE,
                                              DEG_PER_TILE)])

    NSLOT = 3

    @functools.partial(
        pl.kernel,
        out_type=jax.ShapeDtypeStruct((NC, HALF_PAD, H), jnp.float32),
        mesh=mesh,
        scratch_types=[
            pltpu.VMEM((SUP * BLK,), jnp.int32),
            pltpu.VMEM((SUP * BLK,), jnp.int32),
            pltpu.VMEM((SUP, BLK), jnp.int32),
            pltpu.VMEM((SUP, BLK), jnp.int32),
            [pltpu.VMEM((BLK, H), jnp.float32) for _ in range(NSLOT)],
            pltpu.VMEM_SHARED((HALF_PAD, H), jnp.float32),
            [pltpu.SemaphoreType.DMA for _ in range(NSLOT)],
            [pltpu.SemaphoreType.DMA for _ in range(NSLOT)],
        ],
        compiler_params=pltpu.CompilerParams(use_tc_tiling_on_sc=False),
    )
    def sc_edge_sum(edge_hbm, z_hbm, out_hbm, colb, rowb, ccol, crow,
                    bufs, acc_sh, sems_g, sems_s):
        c = lax.axis_index("c")
        s = lax.axis_index("s")
        base = c * HALF
        rb0 = bufs[0]

        _zero_2d(rb0, BLK, H)
        row0 = s * ROWS_PER_TILE
        for k in range(ROWS_PER_TILE // BLK):
            pltpu.sync_copy(rb0, acc_sh.at[_ds8(row0 + k * BLK, BLK), :])
        rem = ROWS_PER_TILE % BLK
        if rem:
            pltpu.sync_copy(
                rb0.at[pl.ds(0, rem), :],
                acc_sh.at[_ds8(row0 + (ROWS_PER_TILE // BLK) * BLK, rem), :])
        plsc.subcore_barrier()

        st = s * BA

        def sup_body(j, _):
            off = (st + j * SUP) * BLK
            pltpu.sync_copy(edge_hbm.at[_ds8(E_PAD + off, SUP * BLK)], colb)
            pltpu.sync_copy(edge_hbm.at[_ds8(off, SUP * BLK)], rowb)
            for b in range(SUP):
                for g in range(BLK // LANES):
                    o = b * BLK + g * LANES
                    cv = colb[pl.ds(o, LANES)]
                    rv = rowb[pl.ds(o, LANES)]
                    inb = (cv >= base) & (cv < base + HALF)
                    crow[b, pl.ds(g * LANES, LANES)] = jnp.where(
                        inb, rv, ZROW + (rv & 63))
                    ccol[b, pl.ds(g * LANES, LANES)] = jnp.where(
                        inb, cv - base, HALF + 8 + (cv & 63))
            g_desc = [None] * NSLOT
            s_desc = [None] * NSLOT
            for b in range(SUP):
                slot = b % NSLOT
                if b >= NSLOT:
                    s_desc[slot].wait()
                g_desc[slot] = pltpu.async_copy(
                    z_hbm.at[crow.at[b]], bufs[slot], sems_g[slot])
                bb = b - (NSLOT - 1)
                if bb >= 0:
                    sl = bb % NSLOT
                    g_desc[sl].wait()
                    s_desc[sl] = pltpu.async_copy(
                        bufs[sl], acc_sh.at[ccol.at[bb]], sems_s[sl],
                        add=True)
            for bb in range(SUP - (NSLOT - 1), SUP):
                sl = bb % NSLOT
                g_desc[sl].wait()
                s_desc[sl] = pltpu.async_copy(
                    bufs[sl], acc_sh.at[ccol.at[bb]], sems_s[sl], add=True)
            for sl in range(NSLOT):
                s_desc[sl].wait()
            return 0

        lax.fori_loop(0, NSUP, sup_body, 0)
        plsc.subcore_barrier()
        for k in range(ROWS_PER_TILE // BLK):
            pltpu.sync_copy(acc_sh.at[_ds8(row0 + k * BLK, BLK), :], rb0)
            pltpu.sync_copy(rb0, out_hbm.at[c, _ds8(row0 + k * BLK, BLK), :])
        if rem:
            r0 = row0 + (ROWS_PER_TILE // BLK) * BLK
            pltpu.sync_copy(acc_sh.at[_ds8(r0, rem), :],
                            rb0.at[pl.ds(0, rem), :])
            pltpu.sync_copy(rb0.at[pl.ds(0, rem), :],
                            out_hbm.at[c, _ds8(r0, rem), :])

    return sc_degree, sc_sscatter, sc_edge_sum


NG = 25
BRS = N // NG

def _row_spec(w):
    return pl.BlockSpec((BRS, w), lambda i: (i, 0))


def _full_spec(shape):
    return pl.BlockSpec(shape, lambda i: (0, 0))


def _tc_pool_body(x_ref, d_ref, win_ref, bin_ref, h_ref, acc):
    i = pl.program_id(0)

    @pl.when(i == 0)
    def _():
        acc[...] = jnp.zeros_like(acc)
        acc[1:2, :] = jnp.full((1, H), -jnp.inf, jnp.float32)

    x = x_ref[...]
    rn = jnp.sqrt(jnp.sum(x * x, axis=1, keepdims=True))
    x = x / jnp.maximum(rn, 1e-12) + d_ref[0, 0]
    acc[0:1, :] = acc[0:1, :] + jnp.sum(x, axis=0, keepdims=True)
    acc[1:2, :] = jnp.maximum(acc[1:2, :], jnp.max(x, axis=0, keepdims=True))
    acc[2:3, :] = acc[2:3, :] + jnp.sum(x * x, axis=0, keepdims=True)

    @pl.when(i == NG - 1)
    def _():
        mean = acc[0:1, :] / N
        mx = acc[1:2, :]
        var1 = (acc[2:3, :] - N * mean * mean) / (N - 1)
        pooled = jnp.concatenate([mean, mx, jnp.sqrt(var1)], axis=1)
        h_ref[...] = jax.nn.relu(
            jnp.dot(pooled, win_ref[...], preferred_element_type=jnp.float32)
            + bin_ref[...])


def _call_pool(x, delta, W_in, b_in, interpret=False):
    return pl.pallas_call(
        _tc_pool_body,
        grid=(NG,),
        in_specs=[_row_spec(H), _full_spec((1, 1)), _full_spec((3 * H, H)),
                  _full_spec((1, H))],
        out_specs=_full_spec((1, H)),
        out_shape=jax.ShapeDtypeStruct((1, H), jnp.float32),
        scratch_shapes=[pltpu.VMEM((8, H), jnp.float32)],
        interpret=interpret,
    )(x, delta, W_in, b_in)


def _tc_dis_body(degp_ref, dis_ref):
    d = degp_ref[...]
    dis_ref[...] = lax.rsqrt(1.0 + d[0:1, :] + d[1:2, :])


def _call_dis(deg_part, interpret=False):
    return pl.pallas_call(
        _tc_dis_body,
        out_shape=jax.ShapeDtypeStruct((1, NP), jnp.float32),
        interpret=interpret,
    )(deg_part)


def _tc_l1stats_body(dis_ref, s_ref, h_ref, wm0_ref, g0_ref, b0_ref,
                     ss_ref, acc):
    i = pl.program_id(0)

    @pl.when(i == 0)
    def _():
        acc[...] = jnp.zeros_like(acc)

    dis = dis_ref[...]
    cvec = dis * (s_ref[...] + dis)
    acc[0:1, 0:1] = acc[0:1, 0:1] + jnp.sum(cvec).reshape(1, 1)
    acc[1:2, 0:1] = acc[1:2, 0:1] + jnp.sum(cvec * cvec).reshape(1, 1)

    @pl.when(i == NG - 1)
    def _():
        mc = acc[0, 0] / N
        vc = acc[1, 0] / N - mc * mc
        m = jnp.dot(h_ref[...], wm0_ref[...],
                    preferred_element_type=jnp.float32)
        scale = m / jnp.sqrt(vc * m * m + 1e-5) * g0_ref[...]
        shift = b0_ref[...] - mc * scale
        ss_ref[0:1, :] = scale
        ss_ref[1:2, :] = shift


def _call_l1stats(dis2d, svec, h, Wm0, g0, b0, interpret=False):
    return pl.pallas_call(
        _tc_l1stats_body,
        grid=(NG,),
        in_specs=[_row_spec(1), _row_spec(1), _full_spec((1, H)),
                  _full_spec((H, H)), _full_spec((1, H)), _full_spec((1, H))],
        out_specs=_full_spec((8, H)),
        out_shape=jax.ShapeDtypeStruct((8, H), jnp.float32),
        scratch_shapes=[pltpu.VMEM((8, H), jnp.float32)],
        interpret=interpret,
    )(dis2d, svec, h, Wm0, g0, b0)


def _tc_l1apply_body(dis_ref, s_ref, ss_ref, wm1_ref, x1_ref, z2_ref):
    dis = dis_ref[...]
    cvec = dis * (s_ref[...] + dis)
    x1 = jax.nn.relu(cvec * ss_ref[0:1, :] + ss_ref[1:2, :])
    x1_ref[...] = x1
    z2_ref[...] = jnp.dot(x1, wm1_ref[...],
                          preferred_element_type=jnp.float32) * dis


def _call_l1apply(dis2d, svec, ss, Wm1, interpret=False):
    return pl.pallas_call(
        _tc_l1apply_body,
        grid=(NG,),
        in_specs=[_row_spec(1), _row_spec(1), _full_spec((8, H)),
                  _full_spec((H, H))],
        out_specs=[_row_spec(H), _row_spec(H)],
        out_shape=[jax.ShapeDtypeStruct((N, H), jnp.float32),
                   jax.ShapeDtypeStruct((N, H), jnp.float32)],
        interpret=interpret,
    )(dis2d, svec, ss, Wm1)


def _tc_lstats_body(x_ref, es_ref, z_ref, dis_ref, ws_ref, cb_ref,
                    out_ref, st_ref, acc):
    i = pl.program_id(0)

    @pl.when(i == 0)
    def _():
        acc[...] = jnp.zeros_like(acc)

    dis = dis_ref[...]
    agg = dis * (es_ref[...] + z_ref[...])
    out = agg + jnp.dot(x_ref[...], ws_ref[...],
                        preferred_element_type=jnp.float32) + cb_ref[...]
    out_ref[...] = out
    acc[0:1, :] = acc[0:1, :] + jnp.sum(out, axis=0, keepdims=True)
    acc[1:2, :] = acc[1:2, :] + jnp.sum(out * out, axis=0, keepdims=True)

    @pl.when(i == NG - 1)
    def _():
        st_ref[...] = acc[...]


def _call_lstats(x, esum, z, dis2d, Ws, cb, interpret=False):
    return pl.pallas_call(
        _tc_lstats_body,
        grid=(NG,),
        in_specs=[_row_spec(H), _row_spec(H), _row_spec(H), _row_spec(1),
                  _full_spec((H, H)), _full_spec((1, H))],
        out_specs=[_row_spec(H), _full_spec((8, H))],
        out_shape=[jax.ShapeDtypeStruct((N, H), jnp.float32),
                   jax.ShapeDtypeStruct((8, H), jnp.float32)],
        scratch_shapes=[pltpu.VMEM((8, H), jnp.float32)],
        interpret=interpret,
    )(x, esum, z, dis2d, Ws, cb)


def _tc_lapply_body(x_ref, out_ref, st_ref, g_ref, b_ref, dis_ref, wmn_ref,
                    xo_ref, zo_ref):
    mean = st_ref[0:1, :] / N
    var = st_ref[1:2, :] / N - mean * mean
    bn = (out_ref[...] - mean) / jnp.sqrt(var + 1e-5) * g_ref[...] + b_ref[...]
    xn = jax.nn.relu(bn) + x_ref[...]
    xo_ref[...] = xn
    zo_ref[...] = jnp.dot(xn, wmn_ref[...],
                          preferred_element_type=jnp.float32) * dis_ref[...]


def _call_lapply(x, out, st, g, b, dis2d, Wmn, interpret=False):
    return pl.pallas_call(
        _tc_lapply_body,
        grid=(NG,),
        in_specs=[_row_spec(H), _row_spec(H), _full_spec((8, H)),
                  _full_spec((1, H)), _full_spec((1, H)), _row_spec(1),
                  _full_spec((H, H))],
        out_specs=[_row_spec(H), _row_spec(H)],
        out_shape=[jax.ShapeDtypeStruct((N, H), jnp.float32),
                   jax.ShapeDtypeStruct((N, H), jnp.float32)],
        interpret=interpret,
    )(x, out, st, g, b, dis2d, Wmn)


def _tc_head_body(x_ref, out_ref, st_ref, g_ref, b_ref, wo_ref, bo_ref,
                  pe_ref):
    mean = st_ref[0:1, :] / N
    var = st_ref[1:2, :] / N - mean * mean
    bn = (out_ref[...] - mean) / jnp.sqrt(var + 1e-5) * g_ref[...] + b_ref[...]
    xn = jax.nn.relu(bn) + x_ref[...]
    pe = jnp.dot(xn, wo_ref[...],
                 preferred_element_type=jnp.float32) + bo_ref[...]
    rn = jnp.sqrt(jnp.sum(pe * pe, axis=1, keepdims=True))
    pe_ref[...] = pe / jnp.maximum(rn, 1e-12)


def _call_head(x, out, st, g, b, W_out, b_out2d, interpret=False):
    P = W_out.shape[1]
    return pl.pallas_call(
        _tc_head_body,
        grid=(NG,),
        in_specs=[_row_spec(H), _row_spec(H), _full_spec((8, H)),
                  _full_spec((1, H)), _full_spec((1, H)),
                  _full_spec((H, P)), _full_spec((1, P))],
        out_specs=_row_spec(P),
        out_shape=jax.ShapeDtypeStruct((N, P), jnp.float32),
        interpret=interpret,
    )(x, out, st, g, b, W_out, b_out2d)


_ZPAD = None


def _pad_z(z):
    return jnp.concatenate(
        [z, jnp.zeros((NP - N, H), jnp.float32)], axis=0)


def kernel(edge_index, num_nodes, x_init, W_in, b_in, W_msg, W_self, conv_b,
           bn_g, bn_b, W_out, b_out):
    f32 = jnp.float32
    sc_degree, sc_sscatter, sc_edge_sum = _sc_kernels()
    delta = (jnp.asarray(num_nodes) - N).astype(f32).reshape(1, 1)

    h = _call_pool(x_init, delta, W_in, b_in.reshape(1, H))

    npad = E_PAD - E
    rowp = jnp.concatenate([edge_index[0],
                            jnp.zeros((npad,), edge_index.dtype)])
    colp = jnp.concatenate([edge_index[1],
                            jnp.full((npad,), N, edge_index.dtype)])
    eflat = jnp.concatenate([rowp, colp])
    deg_part = sc_degree(eflat).reshape(NC, NP)
    dis_flat = _call_dis(deg_part).reshape(NP)
    s_part = sc_sscatter(eflat, dis_flat).reshape(NC, NP)

    dis2d = dis_flat[:N, None]
    svec = (s_part[0, :N] + s_part[1, :N])[:, None]

    ss = _call_l1stats(dis2d, svec, h, W_msg[0], bn_g[0].reshape(1, H),
                       bn_b[0].reshape(1, H))
    x1, z2 = _call_l1apply(dis2d, svec, ss, W_msg[1])

    es2 = sc_edge_sum(eflat, _pad_z(z2))
    esum2 = jnp.concatenate([es2[0, :HALF], es2[1, :HALF]], axis=0)

    out2, st2 = _call_lstats(x1, esum2, z2, dis2d, W_self[1],
                             conv_b[1].reshape(1, H))
    x2, z3 = _call_lapply(x1, out2, st2, bn_g[1].reshape(1, H),
                          bn_b[1].reshape(1, H), dis2d, W_msg[2])

    es3 = sc_edge_sum(eflat, _pad_z(z3))
    esum3 = jnp.concatenate([es3[0, :HALF], es3[1, :HALF]], axis=0)

    out3, st3 = _call_lstats(x2, esum3, z3, dis2d, W_self[2],
                             conv_b[2].reshape(1, H))
    pe = _call_head(x2, out3, st3, bn_g[2].reshape(1, H),
                    bn_b[2].reshape(1, H), W_out,
                    b_out.reshape(1, W_out.shape[1]))
    return pe
